# baseline (speedup 1.0000x reference)
import functools

import jax
import jax.numpy as jnp
from jax import lax
from jax.experimental import pallas as pl
from jax.experimental.pallas import tpu as pltpu
from jax.experimental.pallas import tpu_sc as plsc

NC, NS = 2, 16
NW = NC * NS
L = 16
IPS = 128
NBUF = 2


@functools.partial(jax.jit, static_argnums=(2, 3, 4))
def _sc_gather_t(xt, table, N1, N2, D):
    cols_per_w = N1 // NW
    n_streams = cols_per_w // IPS
    mesh = plsc.VectorSubcoreMesh(core_axis_name="c", subcore_axis_name="s")

    @functools.partial(
        pl.kernel,
        mesh=mesh,
        out_type=jax.ShapeDtypeStruct((N2, D, N1), jnp.float32),
        scratch_types=[
            pltpu.VMEM((NBUF, 1, cols_per_w), jnp.int32),
            pltpu.VMEM((NBUF, cols_per_w, D), jnp.float32),
            pltpu.VMEM((NBUF, 1, D, cols_per_w), jnp.float32),
            [pltpu.SemaphoreType.DMA] * NBUF,
            [pltpu.SemaphoreType.DMA] * NBUF,
            [pltpu.SemaphoreType.DMA] * NBUF,
        ],
        compiler_params=pltpu.CompilerParams(
            use_tc_tiling_on_sc=False, needs_layout_passes=False),
    )
    def k(xt_hbm, table_hbm, out_hbm, idx_v, rows_v, t_v,
          i_sems, g_sems, s_sems):
        wid = lax.axis_index("s") * NC + lax.axis_index("c")
        base_col = wid * cols_per_w

        def idx_copy(i, b):
            return pltpu.make_async_copy(
                xt_hbm.at[pl.ds(i, 1), pl.ds(base_col, cols_per_w)],
                idx_v.at[b], i_sems[b])

        def store_copy(i, b):
            return pltpu.make_async_copy(
                t_v.at[b],
                out_hbm.at[pl.ds(i, 1), pl.ds(0, D),
                           pl.ds(base_col, cols_per_w)],
                s_sems[b])

        def gather_descs(b):
            return [
                pltpu.make_async_copy(
                    table_hbm.at[idx_v.at[b, 0, pl.ds(s * IPS, IPS)]],
                    rows_v.at[b, pl.ds(s * IPS, IPS)],
                    g_sems[b],
                )
                for s in range(n_streams)
            ]

        def transpose(b):
            for g in range(cols_per_w // L):
                rowids = jnp.arange(g * L, (g + 1) * L, dtype=jnp.int32)
                for d in range(D):
                    v = plsc.load_gather(
                        rows_v.at[b], [rowids, jnp.full((L,), d, jnp.int32)])
                    t_v[b, 0, d, pl.ds(g * L, L)] = v

        idx_copy(0, 0).start()
        idx_copy(1, 1).start()
        idx_copy(0, 0).wait()
        for dsc in gather_descs(0):
            dsc.start()

        @pl.loop(0, N2, step=NBUF)
        def _(i0):
            for b in range(NBUF):
                i = i0 + b
                o = 1 - b
                for dsc in gather_descs(b):
                    dsc.wait()
                @pl.when(i + NBUF < N2)
                def _():
                    idx_copy(i + NBUF, b).start()
                @pl.when(i + 1 < N2)
                def _():
                    idx_copy(0, o).wait()
                    for dsc in gather_descs(o):
                        dsc.start()
                @pl.when(i >= NBUF)
                def _():
                    store_copy(0, b).wait()
                transpose(b)
                store_copy(i, b).start()

        for b in range(NBUF):
            store_copy(0, b).wait()

    return k(xt, table)


def kernel(x, weight):
    N1, N2 = x.shape
    D = weight.shape[1]
    xt = x.T.astype(jnp.int32)
    out_t = _sc_gather_t(xt, weight, N1, N2, D)
    return jnp.transpose(out_t, (2, 0, 1))


# speedup vs baseline: 1.5572x; 1.5572x over previous
import functools

import jax
import jax.numpy as jnp
from jax import lax
from jax.experimental import pallas as pl
from jax.experimental.pallas import tpu as pltpu
from jax.experimental.pallas import tpu_sc as plsc

NC, NS = 2, 16
NW = NC * NS
RB = 4
IPS = 40
NBUF = 2


@functools.partial(jax.jit, static_argnums=(2, 3, 4))
def _sc_gather(x, table, N1, N2, D):
    rows_per_w = N1 // NW
    n_chunks = rows_per_w // RB
    assert n_chunks % NBUF == 0 and N2 % IPS == 0
    mesh = plsc.VectorSubcoreMesh(core_axis_name="c", subcore_axis_name="s")

    @functools.partial(
        pl.kernel,
        mesh=mesh,
        out_type=jax.ShapeDtypeStruct((N1, N2, D), jnp.float32),
        scratch_types=[
            pltpu.VMEM((NBUF, RB, N2), jnp.int32),
            pltpu.VMEM((NBUF, RB, N2, D), jnp.float32),
            [pltpu.SemaphoreType.DMA] * NBUF,
            [pltpu.SemaphoreType.DMA] * NBUF,
            [pltpu.SemaphoreType.DMA] * NBUF,
        ],
        compiler_params=pltpu.CompilerParams(use_tc_tiling_on_sc=False),
    )
    def k(x_hbm, table_hbm, out_hbm, idx_v, rows_v, i_sems, g_sems, s_sems):
        wid = lax.axis_index("s") * NC + lax.axis_index("c")
        base_row = wid * rows_per_w

        def idx_copy(i, b):
            return pltpu.make_async_copy(
                x_hbm.at[pl.ds(base_row + i * RB, RB)], idx_v.at[b], i_sems[b])

        def store_copy(i, b):
            return pltpu.make_async_copy(
                rows_v.at[b], out_hbm.at[pl.ds(base_row + i * RB, RB)],
                s_sems[b])

        def gather_descs(b):
            return [
                pltpu.make_async_copy(
                    table_hbm.at[idx_v.at[b, r, pl.ds(h * IPS, IPS)]],
                    rows_v.at[b, r, pl.ds(h * IPS, IPS)],
                    g_sems[b],
                )
                for r in range(RB)
                for h in range(N2 // IPS)
            ]

        idx_copy(0, 0).start()

        @pl.loop(0, n_chunks, step=NBUF)
        def _(i0):
            for b in range(NBUF):
                i = i0 + b
                o = 1 - b
                idx_copy(0, b).wait()
                @pl.when(i >= NBUF)
                def _():
                    store_copy(0, b).wait()
                for d in gather_descs(b):
                    d.start()
                @pl.when(i >= 1)
                def _():
                    for d in gather_descs(o):
                        d.wait()
                    store_copy(i - 1, o).start()
                @pl.when(i + 1 < n_chunks)
                def _():
                    idx_copy(i + 1, o).start()

        last = (n_chunks - 1) % NBUF
        for d in gather_descs(last):
            d.wait()
        store_copy(n_chunks - 1, last).start()
        for b in range(NBUF):
            store_copy(0, b).wait()

    return k(x, table)


def kernel(x, weight):
    N1, N2 = x.shape
    D = weight.shape[1]
    return _sc_gather(x.astype(jnp.int32), weight, N1, N2, D)
